# baseline (speedup 1.0000x reference)
import functools

import jax
import jax.numpy as jnp
from jax import lax
from jax.experimental import pallas as pl
from jax.experimental.pallas import tpu as pltpu
from jax.experimental.pallas import tpu_sc as plsc

_NC = 2
_NS = 16
_NW = _NC * _NS
_IDX_CHUNK = 128
_LANES = 128


def _sc_gather(uidx3, gidx3, utab2, gtab2):
    rows_per_w = uidx3.shape[1]
    bpw = rows_per_w * _IDX_CHUNK
    b = _NW * bpw
    half = rows_per_w // 2

    @functools.partial(
        pl.kernel,
        mesh=plsc.VectorSubcoreMesh(core_axis_name="c", subcore_axis_name="s"),
        out_type=[
            jax.ShapeDtypeStruct((b, _LANES), jnp.float32),
            jax.ShapeDtypeStruct((b, _LANES), jnp.float32),
        ],
        scratch_types=[
            pltpu.VMEM((rows_per_w, _IDX_CHUNK), jnp.int32),
            pltpu.VMEM((rows_per_w, _IDX_CHUNK), jnp.int32),
            pltpu.VMEM((bpw, _LANES), jnp.float32),
            pltpu.VMEM((half * _IDX_CHUNK, _LANES), jnp.float32),
            pltpu.SemaphoreType.DMA,
            pltpu.SemaphoreType.DMA,
        ],
    )
    def gather_kernel(uidx_hbm, gidx_hbm, utab_hbm, gtab_hbm, uout_hbm, gout_hbm,
                      uidx, gidx, ubuf, gbuf, semu, semg):
        wid = lax.axis_index("s") * _NC + lax.axis_index("c")
        pltpu.sync_copy(uidx_hbm.at[wid], uidx)
        pltpu.sync_copy(gidx_hbm.at[wid], gidx)
        ucopies = [
            pltpu.async_copy(utab_hbm.at[uidx.at[j]],
                             ubuf.at[pl.ds(j * _IDX_CHUNK, _IDX_CHUNK)], semu)
            for j in range(rows_per_w)
        ]
        gfirst = [
            pltpu.async_copy(gtab_hbm.at[gidx.at[j]],
                             gbuf.at[pl.ds(j * _IDX_CHUNK, _IDX_CHUNK)], semg)
            for j in range(half)
        ]
        for c in ucopies:
            c.wait()
        pltpu.sync_copy(ubuf, uout_hbm.at[pl.ds(wid * bpw, bpw)])
        for c in gfirst:
            c.wait()
        pltpu.sync_copy(gbuf, gout_hbm.at[pl.ds(wid * bpw, half * _IDX_CHUNK)])
        gsecond = [
            pltpu.async_copy(gtab_hbm.at[gidx.at[half + j]],
                             gbuf.at[pl.ds(j * _IDX_CHUNK, _IDX_CHUNK)], semg)
            for j in range(half)
        ]
        for c in gsecond:
            c.wait()
        pltpu.sync_copy(
            gbuf, gout_hbm.at[pl.ds(wid * bpw + half * _IDX_CHUNK,
                                    half * _IDX_CHUNK)])

    return gather_kernel(uidx3, gidx3, utab2, gtab2)


def _mlp_body(ur_ref, gr_ref, remu_ref, remg_ref, ab_ref, sb_ref, iv_ref,
              xu_ref, xg_ref, at_ref, st_ref,
              w0u_ref, w0g_ref, w0a_ref, w0s_ref, w0i_ref, b0_ref,
              w1_ref, b1_ref, w2_ref, b2_ref, out_ref):
    f32 = jnp.float32
    tile = ur_ref.shape[0]
    dot = functools.partial(jnp.dot, preferred_element_type=f32)

    mu_oh = (remu_ref[...] == lax.broadcasted_iota(jnp.int32, (tile, 8), 1)
             ).astype(f32)
    mg_oh = (remg_ref[...] == lax.broadcasted_iota(jnp.int32, (tile, 16), 1)
             ).astype(f32)
    mu = dot(mu_oh, xu_ref[...])
    mg = dot(mg_oh, xg_ref[...])

    a_onehot = (ab_ref[...] == lax.broadcasted_iota(jnp.int32, (tile, 16), 1)
                ).astype(f32)
    s_onehot = (sb_ref[...] == lax.broadcasted_iota(jnp.int32, (tile, 16), 1)
                ).astype(f32)
    a_fold = dot(at_ref[...], w0a_ref[...])
    s_fold = dot(st_ref[...], w0s_ref[...])

    h = (dot(ur_ref[...] * mu, w0u_ref[...])
         + dot(gr_ref[...] * mg, w0g_ref[...])
         + dot(iv_ref[...], w0i_ref[...])
         + dot(a_onehot, a_fold)
         + dot(s_onehot, s_fold)
         + b0_ref[...])
    h = jnp.maximum(h, 0.0)
    h = jnp.maximum(dot(h, w1_ref[...]) + b1_ref[...], 0.0)
    o = dot(h, w2_ref[...]) + b2_ref[...]
    n2 = jnp.sum(o * o, axis=1, keepdims=True)
    out_ref[...] = o * lax.rsqrt(jnp.maximum(n2, 1e-24))


def _tc_mlp(urows, grows, rem_u, rem_g, age_b, sched_b, interest,
            xu, xg, age_pad, sched_pad,
            W0u_rep, W0g_rep, W0a, W0s, W0i, b0, W1, b1, W2, b2):
    b = urows.shape[0]
    tile = 2048
    grid = (b // tile,)
    d_out = W2.shape[1]

    def rowblk(cols):
        return pl.BlockSpec((tile, cols), lambda i: (i, 0))

    def full(shape):
        return pl.BlockSpec(shape, lambda i: (0, 0))

    return pl.pallas_call(
        _mlp_body,
        grid=grid,
        in_specs=[
            rowblk(_LANES),
            rowblk(_LANES),
            rowblk(1),
            rowblk(1),
            rowblk(1),
            rowblk(1),
            rowblk(interest.shape[1]),
            full(xu.shape),
            full(xg.shape),
            full(age_pad.shape),
            full(sched_pad.shape),
            full(W0u_rep.shape),
            full(W0g_rep.shape),
            full(W0a.shape),
            full(W0s.shape),
            full(W0i.shape),
            full(b0.shape),
            full(W1.shape),
            full(b1.shape),
            full(W2.shape),
            full(b2.shape),
        ],
        out_specs=rowblk(d_out),
        out_shape=jax.ShapeDtypeStruct((b, d_out), jnp.float32),
    )(urows, grows, rem_u, rem_g, age_b, sched_b, interest,
      xu, xg, age_pad, sched_pad,
      W0u_rep, W0g_rep, W0a, W0s, W0i, b0, W1, b1, W2, b2)


def kernel(user_ids, geo_cells, age_buckets, schedule_types, interest_vectors,
           user_table, geo_table, age_table, sched_table,
           W0, b0, W1, b1, W2, b2):
    du = user_table.shape[1]
    dg = geo_table.shape[1]
    upack = _LANES // du
    gpack = _LANES // dg

    uid = user_ids.astype(jnp.int32)
    gid = geo_cells.astype(jnp.int32)
    rows_per_w = uid.shape[0] // (_NW * _IDX_CHUNK)
    uidx3 = (uid // upack).reshape(_NW, rows_per_w, _IDX_CHUNK)
    gidx3 = (gid // gpack).reshape(_NW, rows_per_w, _IDX_CHUNK)
    utab2 = user_table.reshape(-1, _LANES)
    gtab2 = geo_table.reshape(-1, _LANES)
    urows, grows = _sc_gather(uidx3, gidx3, utab2, gtab2)

    da = age_table.shape[1]
    ds_ = sched_table.shape[1]
    di = interest_vectors.shape[1]
    o1 = du
    o2 = o1 + dg
    o3 = o2 + da
    o4 = o3 + ds_
    W0u_rep = jnp.tile(W0[:o1], (upack, 1))
    W0g_rep = jnp.tile(W0[o1:o2], (gpack, 1))
    W0a = W0[o2:o3]
    W0s = W0[o3:o4]
    W0i = W0[o4:o4 + di]

    xu = (jnp.arange(_LANES)[None, :] // du == jnp.arange(8)[:, None]
          ).astype(jnp.float32)
    xg = (jnp.arange(_LANES)[None, :] // dg == jnp.arange(gpack)[:, None]
          ).astype(jnp.float32)

    age_pad = jnp.zeros((16, da), jnp.float32).at[:age_table.shape[0]].set(age_table)
    sched_pad = jnp.zeros((16, ds_), jnp.float32).at[:sched_table.shape[0]].set(sched_table)

    return _tc_mlp(
        urows, grows,
        (uid % upack).reshape(-1, 1),
        (gid % gpack).reshape(-1, 1),
        age_buckets.astype(jnp.int32).reshape(-1, 1),
        schedule_types.astype(jnp.int32).reshape(-1, 1),
        interest_vectors,
        xu, xg, age_pad, sched_pad,
        W0u_rep, W0g_rep, W0a, W0s, W0i,
        b0.reshape(1, -1), W1, b1.reshape(1, -1), W2, b2.reshape(1, -1))


# speedup vs baseline: 1.0052x; 1.0052x over previous
import functools

import jax
import jax.numpy as jnp
from jax import lax
from jax.experimental import pallas as pl
from jax.experimental.pallas import tpu as pltpu
from jax.experimental.pallas import tpu_sc as plsc

_NC = 2
_NS = 16
_NW = _NC * _NS
_IDX_CHUNK = 128
_ROW = 128


def _sc_gather(uidx3, gidx3, utab128, gtab128):
    rows_per_w = uidx3.shape[1]
    bpw = rows_per_w * _IDX_CHUNK
    half = bpw // 2
    b = _NW * bpw

    @functools.partial(
        pl.kernel,
        mesh=plsc.VectorSubcoreMesh(core_axis_name="c", subcore_axis_name="s"),
        out_type=[
            jax.ShapeDtypeStruct((b, _ROW), jnp.float32),
            jax.ShapeDtypeStruct((b, _ROW), jnp.float32),
        ],
        scratch_types=[
            pltpu.VMEM((rows_per_w, _IDX_CHUNK), jnp.int32),
            pltpu.VMEM((rows_per_w, _IDX_CHUNK), jnp.int32),
            pltpu.VMEM((bpw, _ROW), jnp.float32),
            pltpu.VMEM((half, _ROW), jnp.float32),
            pltpu.SemaphoreType.DMA,
            pltpu.SemaphoreType.DMA,
        ],
    )
    def gather_kernel(uidx_hbm, gidx_hbm, utab_hbm, gtab_hbm, uout_hbm, gout_hbm,
                      uidx, gidx, ubuf, gbuf, semu, semg):
        wid = lax.axis_index("s") * _NC + lax.axis_index("c")
        pltpu.sync_copy(uidx_hbm.at[wid], uidx)
        pltpu.sync_copy(gidx_hbm.at[wid], gidx)
        ucopies = [
            pltpu.async_copy(utab_hbm.at[uidx.at[j]],
                             ubuf.at[pl.ds(j * _IDX_CHUNK, _IDX_CHUNK)], semu)
            for j in range(rows_per_w)
        ]
        for wave in range(2):
            gcopies = [
                pltpu.async_copy(
                    gtab_hbm.at[gidx.at[wave * (rows_per_w // 2) + j]],
                    gbuf.at[pl.ds(j * _IDX_CHUNK, _IDX_CHUNK)], semg)
                for j in range(rows_per_w // 2)
            ]
            for c in gcopies:
                c.wait()
            pltpu.sync_copy(gbuf, gout_hbm.at[pl.ds(wid * bpw + wave * half, half)])
        for c in ucopies:
            c.wait()
        pltpu.sync_copy(ubuf, uout_hbm.at[pl.ds(wid * bpw, bpw)])

    return gather_kernel(uidx3, gidx3, utab128, gtab128)


def _mlp_body(ur_ref, gr_ref, remu_ref, remg_ref, ab_ref, sb_ref, iv_ref,
              at_ref, st_ref,
              w0u_ref, w0g_ref, w0a_ref, w0s_ref, w0i_ref, b0_ref,
              w1_ref, b1_ref, w2_ref, b2_ref, out_ref):
    f32 = jnp.float32
    tile = ur_ref.shape[0]
    dot = functools.partial(jnp.dot, preferred_element_type=f32)

    ulane = lax.broadcasted_iota(jnp.int32, (tile, _ROW), 1) // 32
    mu = (ulane == remu_ref[...]).astype(f32)
    glane = lax.broadcasted_iota(jnp.int32, (tile, _ROW), 1) // 8
    mg = (glane == remg_ref[...]).astype(f32)

    a_onehot = (ab_ref[...] == lax.broadcasted_iota(jnp.int32, (tile, 16), 1)
                ).astype(f32)
    s_onehot = (sb_ref[...] == lax.broadcasted_iota(jnp.int32, (tile, 16), 1)
                ).astype(f32)
    a_fold = dot(at_ref[...], w0a_ref[...])
    s_fold = dot(st_ref[...], w0s_ref[...])

    h = (dot(ur_ref[...] * mu, w0u_ref[...])
         + dot(gr_ref[...] * mg, w0g_ref[...])
         + dot(iv_ref[...], w0i_ref[...])
         + dot(a_onehot, a_fold)
         + dot(s_onehot, s_fold)
         + b0_ref[...])
    h = jnp.maximum(h, 0.0)
    h = jnp.maximum(dot(h, w1_ref[...]) + b1_ref[...], 0.0)
    o = dot(h, w2_ref[...]) + b2_ref[...]
    n2 = jnp.sum(o * o, axis=1, keepdims=True)
    out_ref[...] = o * lax.rsqrt(jnp.maximum(n2, 1e-24))


def _tc_mlp(urows, grows, rem_u, rem_g, age_b, sched_b, interest,
            age_pad, sched_pad,
            W0u_t, W0g_t, W0a, W0s, W0i, b0, W1, b1, W2, b2):
    b = urows.shape[0]
    tile = 2048
    grid = (b // tile,)
    d_out = W2.shape[1]

    def rowblk(cols):
        return pl.BlockSpec((tile, cols), lambda i: (i, 0))

    def full(shape):
        return pl.BlockSpec(shape, lambda i: (0, 0))

    return pl.pallas_call(
        _mlp_body,
        grid=grid,
        in_specs=[
            rowblk(_ROW),
            rowblk(_ROW),
            rowblk(1),
            rowblk(1),
            rowblk(1),
            rowblk(1),
            rowblk(interest.shape[1]),
            full(age_pad.shape),
            full(sched_pad.shape),
            full(W0u_t.shape),
            full(W0g_t.shape),
            full(W0a.shape),
            full(W0s.shape),
            full(W0i.shape),
            full(b0.shape),
            full(W1.shape),
            full(b1.shape),
            full(W2.shape),
            full(b2.shape),
        ],
        out_specs=rowblk(d_out),
        out_shape=jax.ShapeDtypeStruct((b, d_out), jnp.float32),
    )(urows, grows, rem_u, rem_g, age_b, sched_b, interest,
      age_pad, sched_pad,
      W0u_t, W0g_t, W0a, W0s, W0i, b0, W1, b1, W2, b2)


def kernel(user_ids, geo_cells, age_buckets, schedule_types, interest_vectors,
           user_table, geo_table, age_table, sched_table,
           W0, b0, W1, b1, W2, b2):
    du = user_table.shape[1]
    dg = geo_table.shape[1]
    upack = _ROW // du
    gpack = _ROW // dg

    uid = user_ids.astype(jnp.int32)
    gid = geo_cells.astype(jnp.int32)
    rows_per_w = uid.shape[0] // (_NW * _IDX_CHUNK)
    uidx3 = (uid // upack).reshape(_NW, rows_per_w, _IDX_CHUNK)
    gidx3 = (gid // gpack).reshape(_NW, rows_per_w, _IDX_CHUNK)
    utab128 = user_table.reshape(-1, _ROW)
    gtab128 = geo_table.reshape(-1, _ROW)
    urows, grows = _sc_gather(uidx3, gidx3, utab128, gtab128)

    da = age_table.shape[1]
    ds_ = sched_table.shape[1]
    di = interest_vectors.shape[1]
    o1 = du
    o2 = o1 + dg
    o3 = o2 + da
    o4 = o3 + ds_
    W0u_t = jnp.tile(W0[:o1], (upack, 1))
    W0g_t = jnp.tile(W0[o1:o2], (gpack, 1))
    W0a = W0[o2:o3]
    W0s = W0[o3:o4]
    W0i = W0[o4:o4 + di]

    age_pad = jnp.zeros((16, da), jnp.float32).at[:age_table.shape[0]].set(age_table)
    sched_pad = jnp.zeros((16, ds_), jnp.float32).at[:sched_table.shape[0]].set(sched_table)

    return _tc_mlp(
        urows, grows,
        (uid % upack).reshape(-1, 1),
        (gid % gpack).reshape(-1, 1),
        age_buckets.astype(jnp.int32).reshape(-1, 1),
        schedule_types.astype(jnp.int32).reshape(-1, 1),
        interest_vectors,
        age_pad, sched_pad,
        W0u_t, W0g_t, W0a, W0s, W0i,
        b0.reshape(1, -1), W1, b1.reshape(1, -1), W2, b2.reshape(1, -1))


# speedup vs baseline: 1.5957x; 1.5874x over previous
import functools

import jax
import jax.numpy as jnp
from jax import lax
from jax.experimental import pallas as pl
from jax.experimental.pallas import tpu as pltpu
from jax.experimental.pallas import tpu_sc as plsc

_NC = 2
_NS = 16
_NW = _NC * _NS
_IDX_CHUNK = 128
_ROW = 128


def _tp_body(x_ref, o_ref, y_ref):
    d, bc = x_ref.shape
    pack = _ROW // d
    y_ref[...] = x_ref[...].T
    o_ref[...] = jnp.concatenate(
        [y_ref[pl.Slice(a, bc // pack, pack), :] for a in range(pack)], axis=1)


def _transpose_pack(tab_t, block_cols):
    d, n = tab_t.shape
    pack = _ROW // d
    grid = (pl.cdiv(n, block_cols),)
    return pl.pallas_call(
        _tp_body,
        grid=grid,
        in_specs=[pl.BlockSpec((d, block_cols), lambda i: (0, i))],
        out_specs=pl.BlockSpec((block_cols // pack, _ROW), lambda i: (i, 0)),
        out_shape=jax.ShapeDtypeStruct((n // pack, _ROW), jnp.float32),
        scratch_shapes=[pltpu.VMEM((block_cols, d), jnp.float32)],
    )(tab_t)


def _sc_gather(uidx3, gidx3, utab128, gtab128):
    rows_per_w = uidx3.shape[1]
    bpw = rows_per_w * _IDX_CHUNK
    half = bpw // 2
    b = _NW * bpw

    @functools.partial(
        pl.kernel,
        mesh=plsc.VectorSubcoreMesh(core_axis_name="c", subcore_axis_name="s"),
        out_type=[
            jax.ShapeDtypeStruct((b, _ROW), jnp.float32),
            jax.ShapeDtypeStruct((b, _ROW), jnp.float32),
        ],
        scratch_types=[
            pltpu.VMEM((rows_per_w, _IDX_CHUNK), jnp.int32),
            pltpu.VMEM((rows_per_w, _IDX_CHUNK), jnp.int32),
            pltpu.VMEM((bpw, _ROW), jnp.float32),
            pltpu.VMEM((half, _ROW), jnp.float32),
            pltpu.SemaphoreType.DMA,
            pltpu.SemaphoreType.DMA,
        ],
    )
    def gather_kernel(uidx_hbm, gidx_hbm, utab_hbm, gtab_hbm, uout_hbm, gout_hbm,
                      uidx, gidx, ubuf, gbuf, semu, semg):
        wid = lax.axis_index("s") * _NC + lax.axis_index("c")
        pltpu.sync_copy(uidx_hbm.at[wid], uidx)
        pltpu.sync_copy(gidx_hbm.at[wid], gidx)
        ucopies = [
            pltpu.async_copy(utab_hbm.at[uidx.at[j]],
                             ubuf.at[pl.ds(j * _IDX_CHUNK, _IDX_CHUNK)], semu)
            for j in range(rows_per_w)
        ]
        for wave in range(2):
            gcopies = [
                pltpu.async_copy(
                    gtab_hbm.at[gidx.at[wave * (rows_per_w // 2) + j]],
                    gbuf.at[pl.ds(j * _IDX_CHUNK, _IDX_CHUNK)], semg)
                for j in range(rows_per_w // 2)
            ]
            for c in gcopies:
                c.wait()
            pltpu.sync_copy(gbuf, gout_hbm.at[pl.ds(wid * bpw + wave * half, half)])
        for c in ucopies:
            c.wait()
        pltpu.sync_copy(ubuf, uout_hbm.at[pl.ds(wid * bpw, bpw)])

    return gather_kernel(uidx3, gidx3, utab128, gtab128)


def _mlp_body(ur_ref, gr_ref, remu_ref, remg_ref, ab_ref, sb_ref, iv_ref,
              at_ref, st_ref,
              w0u_ref, w0g_ref, w0a_ref, w0s_ref, w0i_ref, b0_ref,
              w1_ref, b1_ref, w2_ref, b2_ref, out_ref):
    f32 = jnp.float32
    tile = ur_ref.shape[0]
    dot = functools.partial(jnp.dot, preferred_element_type=f32)

    ulane = lax.broadcasted_iota(jnp.int32, (tile, _ROW), 1) // 32
    mu = (ulane == remu_ref[...]).astype(f32)
    glane = lax.broadcasted_iota(jnp.int32, (tile, _ROW), 1) // 8
    mg = (glane == remg_ref[...]).astype(f32)

    a_onehot = (ab_ref[...] == lax.broadcasted_iota(jnp.int32, (tile, 16), 1)
                ).astype(f32)
    s_onehot = (sb_ref[...] == lax.broadcasted_iota(jnp.int32, (tile, 16), 1)
                ).astype(f32)
    a_fold = dot(at_ref[...], w0a_ref[...])
    s_fold = dot(st_ref[...], w0s_ref[...])

    h = (dot(ur_ref[...] * mu, w0u_ref[...])
         + dot(gr_ref[...] * mg, w0g_ref[...])
         + dot(iv_ref[...], w0i_ref[...])
         + dot(a_onehot, a_fold)
         + dot(s_onehot, s_fold)
         + b0_ref[...])
    h = jnp.maximum(h, 0.0)
    h = jnp.maximum(dot(h, w1_ref[...]) + b1_ref[...], 0.0)
    o = dot(h, w2_ref[...]) + b2_ref[...]
    n2 = jnp.sum(o * o, axis=1, keepdims=True)
    out_ref[...] = o * lax.rsqrt(jnp.maximum(n2, 1e-24))


def _tc_mlp(urows, grows, rem_u, rem_g, age_b, sched_b, interest,
            age_pad, sched_pad,
            W0u_t, W0g_t, W0a, W0s, W0i, b0, W1, b1, W2, b2):
    b = urows.shape[0]
    tile = 2048
    grid = (b // tile,)
    d_out = W2.shape[1]

    def rowblk(cols):
        return pl.BlockSpec((tile, cols), lambda i: (i, 0))

    def full(shape):
        return pl.BlockSpec(shape, lambda i: (0, 0))

    return pl.pallas_call(
        _mlp_body,
        grid=grid,
        in_specs=[
            rowblk(_ROW),
            rowblk(_ROW),
            rowblk(1),
            rowblk(1),
            rowblk(1),
            rowblk(1),
            rowblk(interest.shape[1]),
            full(age_pad.shape),
            full(sched_pad.shape),
            full(W0u_t.shape),
            full(W0g_t.shape),
            full(W0a.shape),
            full(W0s.shape),
            full(W0i.shape),
            full(b0.shape),
            full(W1.shape),
            full(b1.shape),
            full(W2.shape),
            full(b2.shape),
        ],
        out_specs=rowblk(d_out),
        out_shape=jax.ShapeDtypeStruct((b, d_out), jnp.float32),
    )(urows, grows, rem_u, rem_g, age_b, sched_b, interest,
      age_pad, sched_pad,
      W0u_t, W0g_t, W0a, W0s, W0i, b0, W1, b1, W2, b2)


def kernel(user_ids, geo_cells, age_buckets, schedule_types, interest_vectors,
           user_table, geo_table, age_table, sched_table,
           W0, b0, W1, b1, W2, b2):
    du = user_table.shape[1]
    dg = geo_table.shape[1]
    upack = _ROW // du
    gpack = _ROW // dg

    uid = user_ids.astype(jnp.int32)
    gid = geo_cells.astype(jnp.int32)
    rows_per_w = uid.shape[0] // (_NW * _IDX_CHUNK)
    uidx3 = (uid // upack).reshape(_NW, rows_per_w, _IDX_CHUNK)
    gidx3 = (gid // gpack).reshape(_NW, rows_per_w, _IDX_CHUNK)
    utab128 = _transpose_pack(user_table.T, 16384)
    gtab128 = _transpose_pack(geo_table.T, 4096)
    urows, grows = _sc_gather(uidx3, gidx3, utab128, gtab128)

    da = age_table.shape[1]
    ds_ = sched_table.shape[1]
    di = interest_vectors.shape[1]
    o1 = du
    o2 = o1 + dg
    o3 = o2 + da
    o4 = o3 + ds_
    W0u_t = jnp.tile(W0[:o1], (upack, 1))
    W0g_t = jnp.tile(W0[o1:o2], (gpack, 1))
    W0a = W0[o2:o3]
    W0s = W0[o3:o4]
    W0i = W0[o4:o4 + di]

    age_pad = jnp.zeros((16, da), jnp.float32).at[:age_table.shape[0]].set(age_table)
    sched_pad = jnp.zeros((16, ds_), jnp.float32).at[:sched_table.shape[0]].set(sched_table)

    return _tc_mlp(
        urows, grows,
        (uid % upack).reshape(-1, 1),
        (gid % gpack).reshape(-1, 1),
        age_buckets.astype(jnp.int32).reshape(-1, 1),
        schedule_types.astype(jnp.int32).reshape(-1, 1),
        interest_vectors,
        age_pad, sched_pad,
        W0u_t, W0g_t, W0a, W0s, W0i,
        b0.reshape(1, -1), W1, b1.reshape(1, -1), W2, b2.reshape(1, -1))
